# baseline (speedup 1.0000x reference)
import jax
import jax.numpy as jnp
from jax.experimental import pallas as pl
from jax.experimental.pallas import tpu as pltpu

NUM_EXPERTS = 8
IN_FEATURES = 1024
OUT_FEATURES = 1024
N_TOKENS = 8192
BLOCK_N = 1024


def _moe_body(x_ref, gw_ref, gb_ref, ew_ref, eb_ref, out_ref):
    x = x_ref[...]
    logits = (
        jnp.dot(x, gw_ref[...], preferred_element_type=jnp.float32) + gb_ref[...]
    )
    m = jnp.max(logits, axis=-1, keepdims=True)
    ex = jnp.exp(logits - m)
    s = ex / jnp.sum(ex, axis=-1, keepdims=True)
    acc = jnp.dot(s, eb_ref[...], preferred_element_type=jnp.float32)
    for e in range(NUM_EXPERTS):
        xe = (s[:, e : e + 1] * x).astype(jnp.bfloat16)
        acc = acc + jnp.dot(xe, ew_ref[e], preferred_element_type=jnp.float32)
    out_ref[...] = acc


@jax.jit
def kernel(x, gate_W, gate_b, expert_W, expert_b):
    n_blocks = N_TOKENS // BLOCK_N
    out = pl.pallas_call(
        _moe_body,
        grid=(n_blocks,),
        in_specs=[
            pl.BlockSpec((BLOCK_N, IN_FEATURES), lambda i: (i, 0)),
            pl.BlockSpec((IN_FEATURES, NUM_EXPERTS), lambda i: (0, 0)),
            pl.BlockSpec((1, NUM_EXPERTS), lambda i: (0, 0)),
            pl.BlockSpec(
                (NUM_EXPERTS, IN_FEATURES, OUT_FEATURES), lambda i: (0, 0, 0)
            ),
            pl.BlockSpec((NUM_EXPERTS, OUT_FEATURES), lambda i: (0, 0)),
        ],
        out_specs=pl.BlockSpec((BLOCK_N, OUT_FEATURES), lambda i: (i, 0)),
        out_shape=jax.ShapeDtypeStruct((N_TOKENS, OUT_FEATURES), jnp.float32),
        compiler_params=pltpu.CompilerParams(
            dimension_semantics=("arbitrary",),
        ),
    )(x, gate_W, gate_b.reshape(1, NUM_EXPERTS), expert_W.astype(jnp.bfloat16), expert_b)
    return out


# speedup vs baseline: 1.0033x; 1.0033x over previous
import jax
import jax.numpy as jnp
from jax.experimental import pallas as pl
from jax.experimental.pallas import tpu as pltpu

NUM_EXPERTS = 8
IN_FEATURES = 1024
OUT_FEATURES = 1024
N_TOKENS = 8192
BLOCK_N = 1024
EXPERTS_PER_CHUNK = 2
NUM_CHUNKS = NUM_EXPERTS // EXPERTS_PER_CHUNK
CHUNK_K = EXPERTS_PER_CHUNK * IN_FEATURES


def _moe_body(x_ref, gw_ref, gb_ref, ew_ref, eb_ref, out_ref, *xs_refs):
    x = x_ref[...]
    logits = (
        jnp.dot(x, gw_ref[...], preferred_element_type=jnp.float32) + gb_ref[...]
    )
    m = jnp.max(logits, axis=-1, keepdims=True)
    ex = jnp.exp(logits - m)
    s = ex / jnp.sum(ex, axis=-1, keepdims=True)
    acc = jnp.dot(s, eb_ref[...], preferred_element_type=jnp.float32)
    partials = []
    for c in range(NUM_CHUNKS):
        for j in range(EXPERTS_PER_CHUNK):
            e = c * EXPERTS_PER_CHUNK + j
            xs_refs[c][:, j * IN_FEATURES : (j + 1) * IN_FEATURES] = (
                s[:, e : e + 1] * x
            ).astype(jnp.bfloat16)
        partials.append(
            jnp.dot(
                xs_refs[c][...],
                ew_ref[c * CHUNK_K : (c + 1) * CHUNK_K, :],
                preferred_element_type=jnp.float32,
            )
        )
    for p in partials:
        acc = acc + p
    out_ref[...] = acc


@jax.jit
def kernel(x, gate_W, gate_b, expert_W, expert_b):
    n_blocks = N_TOKENS // BLOCK_N
    ew = expert_W.reshape(NUM_EXPERTS * IN_FEATURES, OUT_FEATURES).astype(
        jnp.bfloat16
    )
    out = pl.pallas_call(
        _moe_body,
        grid=(n_blocks,),
        in_specs=[
            pl.BlockSpec((BLOCK_N, IN_FEATURES), lambda i: (i, 0)),
            pl.BlockSpec((IN_FEATURES, NUM_EXPERTS), lambda i: (0, 0)),
            pl.BlockSpec((1, NUM_EXPERTS), lambda i: (0, 0)),
            pl.BlockSpec((NUM_EXPERTS * IN_FEATURES, OUT_FEATURES), lambda i: (0, 0)),
            pl.BlockSpec((NUM_EXPERTS, OUT_FEATURES), lambda i: (0, 0)),
        ],
        out_specs=pl.BlockSpec((BLOCK_N, OUT_FEATURES), lambda i: (i, 0)),
        out_shape=jax.ShapeDtypeStruct((N_TOKENS, OUT_FEATURES), jnp.float32),
        scratch_shapes=[
            pltpu.VMEM((BLOCK_N, CHUNK_K), jnp.bfloat16)
            for _ in range(NUM_CHUNKS)
        ],
        compiler_params=pltpu.CompilerParams(
            dimension_semantics=("arbitrary",),
        ),
    )(x, gate_W, gate_b.reshape(1, NUM_EXPERTS), ew, expert_b)
    return out
